# baseline (speedup 1.0000x reference)
import jax
import jax.numpy as jnp
from jax import lax
from jax.experimental import pallas as pl
from jax.experimental.pallas import tpu as pltpu
from jax.experimental.pallas import tpu_sc as plsc

D = 224
N = 32
R = D * D
CHUNK = 256
NSUB = CHUNK // 128
NCHUNK = R // CHUNK
NG = CHUNK // 16

_LO = 0.001
_HI = D - 1.001


def _axis_terms(qo, c_chunk, sign):
    q = qo + sign * c_chunk
    qc = jnp.minimum(jnp.maximum(q, _LO), _HI)
    fi = qc.astype(jnp.int32)
    ff = fi.astype(jnp.float32)
    frac_pos = qc > ff
    cf = ff + jnp.where(frac_pos, 1.0, 0.0)
    wf = 1.0 - (qc - ff)
    wc = 1.0 - (cf - qc)
    ca = wf + jnp.where(frac_pos, 0.0, wc)
    cb = jnp.where(frac_pos, wc, 0.0)
    d = q - qc
    return fi, ca, cb, d * d


def _sc_body(t1, t2, cfl, m1f, m2f, out, lossv,
             auxv0, auxv1, ia0, ia1, ib0, ib1,
             cf0, cf1, g0, g1, outv0, outv1, accs,
             semg0, semg1, semo0, semo1):
    wid = lax.axis_index("s") * 2 + lax.axis_index("c")
    n = wid
    iot = lax.iota(jnp.int32, 16)
    nR = n * R
    auxs = (auxv0, auxv1)
    ias = (ia0, ia1)
    ibs = (ib0, ib1)
    cfs = (cf0, cf1)
    gs = (g0, g1)
    outvs = (outv0, outv1)
    semgs = (semg0, semg1)
    semos = (semo0, semo1)

    def phase1(g, b, acc):
        auxv, ia, ib, cf = auxs[b], ias[b], ibs[b], cfs[b]
        base = g * CHUNK
        pltpu.sync_copy(cfl.at[pl.ds(n * 2 * R + base, CHUNK)],
                        auxv.at[pl.ds(0, CHUNK)])
        pltpu.sync_copy(cfl.at[pl.ds(n * 2 * R + R + base, CHUNK)],
                        auxv.at[pl.ds(CHUNK, CHUNK)])
        pltpu.sync_copy(m1f.at[pl.ds(n * R + base, CHUNK)],
                        auxv.at[pl.ds(2 * CHUNK, CHUNK)])
        pltpu.sync_copy(m2f.at[pl.ds(n * R + base, CHUNK)],
                        auxv.at[pl.ds(3 * CHUNK, CHUNK)])
        for g2 in range(NG):
            s = g2 * 16
            j, sj = divmod(s, 128)
            c0 = auxv[pl.ds(s, 16)]
            c1 = auxv[pl.ds(CHUNK + s, 16)]
            p = base + s + iot
            q0 = lax.div(p, D).astype(jnp.float32)
            q1 = lax.rem(p, D).astype(jnp.float32)
            f0, ra, rb, d0 = _axis_terms(q0, c0, 1.0)
            f1, cca, ccb, d1 = _axis_terms(q1, c1, 1.0)
            idx = nR + f0 * D + f1
            ia[j, pl.ds(sj, 16)] = idx
            ia[NSUB + j, pl.ds(sj, 16)] = idx + D
            cf[0, pl.ds(s, 16)] = ra
            cf[1, pl.ds(s, 16)] = rb
            cf[2, pl.ds(s, 16)] = cca
            cf[3, pl.ds(s, 16)] = ccb
            acc = acc + d0 + d1
            f0, ra, rb, d0 = _axis_terms(q0, c0, -1.0)
            f1, cca, ccb, d1 = _axis_terms(q1, c1, -1.0)
            idx = nR + f0 * D + f1
            ib[j, pl.ds(sj, 16)] = idx
            ib[NSUB + j, pl.ds(sj, 16)] = idx + D
            cf[4, pl.ds(s, 16)] = ra
            cf[5, pl.ds(s, 16)] = rb
            cf[6, pl.ds(s, 16)] = cca
            cf[7, pl.ds(s, 16)] = ccb
            acc = acc + d0 + d1
        gb, sg = gs[b], semgs[b]
        for j in range(NSUB):
            pltpu.async_copy(t1.at[ia.at[j]],
                             gb.at[pl.ds(j * 128, 128)], sg)
            pltpu.async_copy(t1.at[ia.at[NSUB + j]],
                             gb.at[pl.ds(CHUNK + j * 128, 128)], sg)
            pltpu.async_copy(t2.at[ib.at[j]],
                             gb.at[pl.ds(2 * CHUNK + j * 128, 128)], sg)
            pltpu.async_copy(t2.at[ib.at[NSUB + j]],
                             gb.at[pl.ds(3 * CHUNK + j * 128, 128)], sg)
        return acc

    def wait_gathers(b):
        gb, sg = gs[b], semgs[b]
        for j in range(NSUB):
            for r in range(4):
                pltpu.make_async_copy(
                    t1.at[ias[b].at[j]],
                    gb.at[pl.ds(r * CHUNK + j * 128, 128)], sg).wait()

    def phase2(g, b):
        auxv, cf, gb, outv = auxs[b], cfs[b], gs[b], outvs[b]
        base = g * CHUNK
        @pl.when(g >= 2)
        def _():
            for ch in range(3):
                pltpu.make_async_copy(
                    outv.at[pl.ds(ch * CHUNK, CHUNK)],
                    out.at[pl.ds(ch * CHUNK, CHUNK)], semos[b]).wait()
        for g2 in range(NG):
            s = g2 * 16
            rows = s + iot
            m1 = auxv[pl.ds(2 * CHUNK + s, 16)]
            m2 = auxv[pl.ds(3 * CHUNK + s, 16)]
            ra1 = cf[0, pl.ds(s, 16)]
            rb1 = cf[1, pl.ds(s, 16)]
            ca1 = cf[2, pl.ds(s, 16)]
            cb1 = cf[3, pl.ds(s, 16)]
            ra2 = cf[4, pl.ds(s, 16)]
            rb2 = cf[5, pl.ds(s, 16)]
            ca2 = cf[6, pl.ds(s, 16)]
            cb2 = cf[7, pl.ds(s, 16)]
            for ch in range(3):
                c_lo = jnp.full((16,), ch, jnp.int32)
                c_hi = jnp.full((16,), ch + 3, jnp.int32)
                r1f = rows
                r1c = rows + CHUNK
                r2f = rows + 2 * CHUNK
                r2c = rows + 3 * CHUNK
                v1 = (ra1 * (ca1 * plsc.load_gather(gb, [r1f, c_lo])
                             + cb1 * plsc.load_gather(gb, [r1f, c_hi]))
                      + rb1 * (ca1 * plsc.load_gather(gb, [r1c, c_lo])
                               + cb1 * plsc.load_gather(gb, [r1c, c_hi])))
                v2 = (ra2 * (ca2 * plsc.load_gather(gb, [r2f, c_lo])
                             + cb2 * plsc.load_gather(gb, [r2f, c_hi]))
                      + rb2 * (ca2 * plsc.load_gather(gb, [r2c, c_lo])
                               + cb2 * plsc.load_gather(gb, [r2c, c_hi])))
                outv[pl.ds(ch * CHUNK + s, 16)] = v1 * m1 + v2 * m2
        for ch in range(3):
            pltpu.async_copy(
                outv.at[pl.ds(ch * CHUNK, CHUNK)],
                out.at[pl.ds((n * 3 + ch) * R + base, CHUNK)], semos[b])

    acc0 = phase1(0, 0, jnp.zeros((16,), jnp.float32))

    def pair(k, acc):
        for b in range(2):
            g = 2 * k + b
            acc = lax.cond(
                g + 1 < NCHUNK,
                lambda a: phase1(g + 1, 1 - b, a),
                lambda a: a,
                acc)
            wait_gathers(b)
            phase2(g, b)
        return acc

    acc = lax.fori_loop(0, NCHUNK // 2, pair, acc0)
    for b in range(2):
        for ch in range(3):
            pltpu.make_async_copy(
                outvs[b].at[pl.ds(ch * CHUNK, CHUNK)],
                out.at[pl.ds(ch * CHUNK, CHUNK)], semos[b]).wait()
    accs[...] = acc
    pltpu.sync_copy(accs, lossv.at[pl.ds(wid * 16, 16)])


@jax.jit
def _run(t1, t2, cfl, m1f, m2f):
    mesh = plsc.VectorSubcoreMesh(core_axis_name="c", subcore_axis_name="s")
    f = pl.kernel(
        _sc_body,
        out_type=[
            jax.ShapeDtypeStruct((N * 3 * R,), jnp.float32),
            jax.ShapeDtypeStruct((N * 16,), jnp.float32),
        ],
        mesh=mesh,
        compiler_params=pltpu.CompilerParams(
            needs_layout_passes=False, use_tc_tiling_on_sc=False),
        scratch_types=[
            pltpu.VMEM((4 * CHUNK,), jnp.float32),
            pltpu.VMEM((4 * CHUNK,), jnp.float32),
            pltpu.VMEM((2 * NSUB, 128), jnp.int32),
            pltpu.VMEM((2 * NSUB, 128), jnp.int32),
            pltpu.VMEM((2 * NSUB, 128), jnp.int32),
            pltpu.VMEM((2 * NSUB, 128), jnp.int32),
            pltpu.VMEM((8, CHUNK), jnp.float32),
            pltpu.VMEM((8, CHUNK), jnp.float32),
            pltpu.VMEM((4 * CHUNK, 8), jnp.float32),
            pltpu.VMEM((4 * CHUNK, 8), jnp.float32),
            pltpu.VMEM((3 * CHUNK,), jnp.float32),
            pltpu.VMEM((3 * CHUNK,), jnp.float32),
            pltpu.VMEM((16,), jnp.float32),
            pltpu.SemaphoreType.DMA,
            pltpu.SemaphoreType.DMA,
            pltpu.SemaphoreType.DMA,
            pltpu.SemaphoreType.DMA,
        ],
    )
    return f(t1, t2, cfl, m1f, m2f)


def _pair_table(im):
    cl = jnp.transpose(im, (0, 2, 3, 1))
    nxt = jnp.roll(cl, -1, axis=2)
    t = jnp.concatenate(
        [cl, nxt, jnp.zeros((N, D, D, 2), cl.dtype)], axis=-1)
    return t.reshape(N * R, 8)


def kernel(im1, im2, C, M1, M2):
    t1 = _pair_table(im1)
    t2 = _pair_table(im2)
    cf = C.reshape(N * 2 * R)
    m1f = M1.reshape(N * R)
    m2f = M2.reshape(N * R)
    out_flat, loss_part = _run(t1, t2, cf, m1f, m2f)
    out = out_flat.reshape(N, 3, D, D)
    loss = jnp.sum(loss_part) * (0.01 / (N * 2.0 * R * D * D))
    return out, loss


# speedup vs baseline: 1.0978x; 1.0978x over previous
import jax
import jax.numpy as jnp
from jax import lax
from jax.experimental import pallas as pl
from jax.experimental.pallas import tpu as pltpu
from jax.experimental.pallas import tpu_sc as plsc

D = 224
N = 32
R = D * D
CHUNK = 256
NSUB = CHUNK // 128
NCHUNK = R // CHUNK
NG = CHUNK // 16

_LO = 0.001
_HI = D - 1.001


def _axis_terms(qo, c_chunk, sign):
    q = qo + sign * c_chunk
    qc = jnp.minimum(jnp.maximum(q, _LO), _HI)
    fi = qc.astype(jnp.int32)
    ff = fi.astype(jnp.float32)
    frac_pos = qc > ff
    cf = ff + jnp.where(frac_pos, 1.0, 0.0)
    wf = 1.0 - (qc - ff)
    wc = 1.0 - (cf - qc)
    ca = wf + jnp.where(frac_pos, 0.0, wc)
    cb = jnp.where(frac_pos, wc, 0.0)
    d = q - qc
    return fi, ca, cb, d * d


def _sc_body(t1, t2, cfl, m1f, m2f, out, lossv,
             auxv0, auxv1, ia0, ia1, ib0, ib1,
             cf0, cf1, g0, g1, outv0, outv1, accs,
             semg0, semg1, semo0, semo1, sema0, sema1):
    wid = lax.axis_index("s") * 2 + lax.axis_index("c")
    n = wid
    iot = lax.iota(jnp.int32, 16)
    nR = n * R
    auxs = (auxv0, auxv1)
    ias = (ia0, ia1)
    ibs = (ib0, ib1)
    cfs = (cf0, cf1)
    gs = (g0, g1)
    outvs = (outv0, outv1)
    semgs = (semg0, semg1)
    semos = (semo0, semo1)
    semas = (sema0, sema1)

    def aux_copies(g, b):
        auxv, sa = auxs[b], semas[b]
        base = g * CHUNK
        return (
            (cfl.at[pl.ds(n * 2 * R + base, CHUNK)],
             auxv.at[pl.ds(0, CHUNK)], sa),
            (cfl.at[pl.ds(n * 2 * R + R + base, CHUNK)],
             auxv.at[pl.ds(CHUNK, CHUNK)], sa),
            (m1f.at[pl.ds(n * R + base, CHUNK)],
             auxv.at[pl.ds(2 * CHUNK, CHUNK)], sa),
            (m2f.at[pl.ds(n * R + base, CHUNK)],
             auxv.at[pl.ds(3 * CHUNK, CHUNK)], sa),
        )

    def fire_aux(g, b):
        for src, dst, sa in aux_copies(g, b):
            pltpu.async_copy(src, dst, sa)

    def wait_aux(g, b):
        for src, dst, sa in aux_copies(g, b):
            pltpu.make_async_copy(src, dst, sa).wait()

    def phase1(g, b, acc):
        auxv, ia, ib, cf = auxs[b], ias[b], ibs[b], cfs[b]
        base = g * CHUNK
        wait_aux(g, b)
        for g2 in range(NG):
            s = g2 * 16
            j, sj = divmod(s, 128)
            c0 = auxv[pl.ds(s, 16)]
            c1 = auxv[pl.ds(CHUNK + s, 16)]
            p = base + s + iot
            q0 = lax.div(p, D).astype(jnp.float32)
            q1 = lax.rem(p, D).astype(jnp.float32)
            f0, ra, rb, d0 = _axis_terms(q0, c0, 1.0)
            f1, cca, ccb, d1 = _axis_terms(q1, c1, 1.0)
            idx = nR + f0 * D + f1
            ia[j, pl.ds(sj, 16)] = idx
            ia[NSUB + j, pl.ds(sj, 16)] = idx + D
            cf[0, pl.ds(s, 16)] = ra
            cf[1, pl.ds(s, 16)] = rb
            cf[2, pl.ds(s, 16)] = cca
            cf[3, pl.ds(s, 16)] = ccb
            acc = acc + d0 + d1
            f0, ra, rb, d0 = _axis_terms(q0, c0, -1.0)
            f1, cca, ccb, d1 = _axis_terms(q1, c1, -1.0)
            idx = nR + f0 * D + f1
            ib[j, pl.ds(sj, 16)] = idx
            ib[NSUB + j, pl.ds(sj, 16)] = idx + D
            cf[4, pl.ds(s, 16)] = ra
            cf[5, pl.ds(s, 16)] = rb
            cf[6, pl.ds(s, 16)] = cca
            cf[7, pl.ds(s, 16)] = ccb
            acc = acc + d0 + d1
        gb, sg = gs[b], semgs[b]
        for j in range(NSUB):
            pltpu.async_copy(t1.at[ia.at[j]],
                             gb.at[pl.ds(j * 128, 128)], sg)
            pltpu.async_copy(t1.at[ia.at[NSUB + j]],
                             gb.at[pl.ds(CHUNK + j * 128, 128)], sg)
            pltpu.async_copy(t2.at[ib.at[j]],
                             gb.at[pl.ds(2 * CHUNK + j * 128, 128)], sg)
            pltpu.async_copy(t2.at[ib.at[NSUB + j]],
                             gb.at[pl.ds(3 * CHUNK + j * 128, 128)], sg)
        return acc

    def wait_gathers(b):
        gb, sg = gs[b], semgs[b]
        for j in range(NSUB):
            for r in range(4):
                pltpu.make_async_copy(
                    t1.at[ias[b].at[j]],
                    gb.at[pl.ds(r * CHUNK + j * 128, 128)], sg).wait()

    def phase2(g, b):
        auxv, cf, gb, outv = auxs[b], cfs[b], gs[b], outvs[b]
        base = g * CHUNK
        @pl.when(g >= 2)
        def _():
            for ch in range(3):
                pltpu.make_async_copy(
                    outv.at[pl.ds(ch * CHUNK, CHUNK)],
                    out.at[pl.ds(ch * CHUNK, CHUNK)], semos[b]).wait()
        for g2 in range(NG):
            s = g2 * 16
            rows = s + iot
            m1 = auxv[pl.ds(2 * CHUNK + s, 16)]
            m2 = auxv[pl.ds(3 * CHUNK + s, 16)]
            ra1 = cf[0, pl.ds(s, 16)]
            rb1 = cf[1, pl.ds(s, 16)]
            ca1 = cf[2, pl.ds(s, 16)]
            cb1 = cf[3, pl.ds(s, 16)]
            ra2 = cf[4, pl.ds(s, 16)]
            rb2 = cf[5, pl.ds(s, 16)]
            ca2 = cf[6, pl.ds(s, 16)]
            cb2 = cf[7, pl.ds(s, 16)]
            for ch in range(3):
                c_lo = jnp.full((16,), ch, jnp.int32)
                c_hi = jnp.full((16,), ch + 3, jnp.int32)
                r1f = rows
                r1c = rows + CHUNK
                r2f = rows + 2 * CHUNK
                r2c = rows + 3 * CHUNK
                v1 = (ra1 * (ca1 * plsc.load_gather(gb, [r1f, c_lo])
                             + cb1 * plsc.load_gather(gb, [r1f, c_hi]))
                      + rb1 * (ca1 * plsc.load_gather(gb, [r1c, c_lo])
                               + cb1 * plsc.load_gather(gb, [r1c, c_hi])))
                v2 = (ra2 * (ca2 * plsc.load_gather(gb, [r2f, c_lo])
                             + cb2 * plsc.load_gather(gb, [r2f, c_hi]))
                      + rb2 * (ca2 * plsc.load_gather(gb, [r2c, c_lo])
                               + cb2 * plsc.load_gather(gb, [r2c, c_hi])))
                outv[pl.ds(ch * CHUNK + s, 16)] = v1 * m1 + v2 * m2
        for ch in range(3):
            pltpu.async_copy(
                outv.at[pl.ds(ch * CHUNK, CHUNK)],
                out.at[pl.ds((n * 3 + ch) * R + base, CHUNK)], semos[b])

    fire_aux(0, 0)
    acc0 = phase1(0, 0, jnp.zeros((16,), jnp.float32))
    fire_aux(1, 1)

    def pair(k, acc):
        for b in range(2):
            g = 2 * k + b
            acc = lax.cond(
                g + 1 < NCHUNK,
                lambda a: phase1(g + 1, 1 - b, a),
                lambda a: a,
                acc)
            wait_gathers(b)
            phase2(g, b)

            @pl.when(g + 2 < NCHUNK)
            def _():
                fire_aux(g + 2, b)
        return acc

    acc = lax.fori_loop(0, NCHUNK // 2, pair, acc0)
    for b in range(2):
        for ch in range(3):
            pltpu.make_async_copy(
                outvs[b].at[pl.ds(ch * CHUNK, CHUNK)],
                out.at[pl.ds(ch * CHUNK, CHUNK)], semos[b]).wait()
    accs[...] = acc
    pltpu.sync_copy(accs, lossv.at[pl.ds(wid * 16, 16)])


@jax.jit
def _run(t1, t2, cfl, m1f, m2f):
    mesh = plsc.VectorSubcoreMesh(core_axis_name="c", subcore_axis_name="s")
    f = pl.kernel(
        _sc_body,
        out_type=[
            jax.ShapeDtypeStruct((N * 3 * R,), jnp.float32),
            jax.ShapeDtypeStruct((N * 16,), jnp.float32),
        ],
        mesh=mesh,
        compiler_params=pltpu.CompilerParams(
            needs_layout_passes=False, use_tc_tiling_on_sc=False),
        scratch_types=[
            pltpu.VMEM((4 * CHUNK,), jnp.float32),
            pltpu.VMEM((4 * CHUNK,), jnp.float32),
            pltpu.VMEM((2 * NSUB, 128), jnp.int32),
            pltpu.VMEM((2 * NSUB, 128), jnp.int32),
            pltpu.VMEM((2 * NSUB, 128), jnp.int32),
            pltpu.VMEM((2 * NSUB, 128), jnp.int32),
            pltpu.VMEM((8, CHUNK), jnp.float32),
            pltpu.VMEM((8, CHUNK), jnp.float32),
            pltpu.VMEM((4 * CHUNK, 8), jnp.float32),
            pltpu.VMEM((4 * CHUNK, 8), jnp.float32),
            pltpu.VMEM((3 * CHUNK,), jnp.float32),
            pltpu.VMEM((3 * CHUNK,), jnp.float32),
            pltpu.VMEM((16,), jnp.float32),
            pltpu.SemaphoreType.DMA,
            pltpu.SemaphoreType.DMA,
            pltpu.SemaphoreType.DMA,
            pltpu.SemaphoreType.DMA,
            pltpu.SemaphoreType.DMA,
            pltpu.SemaphoreType.DMA,
        ],
    )
    return f(t1, t2, cfl, m1f, m2f)


def _pair_table(im):
    cl = jnp.transpose(im, (0, 2, 3, 1))
    nxt = jnp.roll(cl, -1, axis=2)
    t = jnp.concatenate(
        [cl, nxt, jnp.zeros((N, D, D, 2), cl.dtype)], axis=-1)
    return t.reshape(N * R, 8)


def kernel(im1, im2, C, M1, M2):
    t1 = _pair_table(im1)
    t2 = _pair_table(im2)
    cf = C.reshape(N * 2 * R)
    m1f = M1.reshape(N * R)
    m2f = M2.reshape(N * R)
    out_flat, loss_part = _run(t1, t2, cf, m1f, m2f)
    out = out_flat.reshape(N, 3, D, D)
    loss = jnp.sum(loss_part) * (0.01 / (N * 2.0 * R * D * D))
    return out, loss


# speedup vs baseline: 2.8544x; 2.6002x over previous
import jax
import jax.numpy as jnp
from jax import lax
from jax.experimental import pallas as pl
from jax.experimental.pallas import tpu as pltpu
from jax.experimental.pallas import tpu_sc as plsc

D = 224
N = 32
R = D * D
CHUNK = 256
NSUB = CHUNK // 128
NCHUNK = R // CHUNK
NG = CHUNK // 16

_LO = 0.001
_HI = D - 1.001


def _axis_terms(qo, c_chunk, sign):
    q = qo + sign * c_chunk
    qc = jnp.minimum(jnp.maximum(q, _LO), _HI)
    fi = qc.astype(jnp.int32)
    ff = fi.astype(jnp.float32)
    frac_pos = qc > ff
    cf = ff + jnp.where(frac_pos, 1.0, 0.0)
    wf = 1.0 - (qc - ff)
    wc = 1.0 - (cf - qc)
    ca = wf + jnp.where(frac_pos, 0.0, wc)
    cb = jnp.where(frac_pos, wc, 0.0)
    d = q - qc
    return fi, ca, cb, d * d


BLOCK = 16 * D
NBLK = R // BLOCK


def _sc_body(im1f, im2f, cfl, m1f, m2f, out, lossv, t1, t2,
             auxv0, auxv1, ia0, ia1, ib0, ib1,
             cf0, cf1, g0, g1, outv0, outv1, accs,
             stg0, stg1, stg2, ob,
             semg0, semg1, semo0, semo1, sema0, sema1, semb):
    wid = lax.axis_index("s") * 2 + lax.axis_index("c")
    n = wid
    iot = lax.iota(jnp.int32, 16)
    nR = n * R
    auxs = (auxv0, auxv1)
    ias = (ia0, ia1)
    ibs = (ib0, ib1)
    cfs = (cf0, cf1)
    gs = (g0, g1)
    outvs = (outv0, outv1)
    semgs = (semg0, semg1)
    semos = (semo0, semo1)
    semas = (sema0, sema1)

    def aux_copies(g, b):
        auxv, sa = auxs[b], semas[b]
        base = g * CHUNK
        return (
            (cfl.at[pl.ds(n * 2 * R + base, CHUNK)],
             auxv.at[pl.ds(0, CHUNK)], sa),
            (cfl.at[pl.ds(n * 2 * R + R + base, CHUNK)],
             auxv.at[pl.ds(CHUNK, CHUNK)], sa),
            (m1f.at[pl.ds(n * R + base, CHUNK)],
             auxv.at[pl.ds(2 * CHUNK, CHUNK)], sa),
            (m2f.at[pl.ds(n * R + base, CHUNK)],
             auxv.at[pl.ds(3 * CHUNK, CHUNK)], sa),
        )

    def fire_aux(g, b):
        for src, dst, sa in aux_copies(g, b):
            pltpu.async_copy(src, dst, sa)

    def wait_aux(g, b):
        for src, dst, sa in aux_copies(g, b):
            pltpu.make_async_copy(src, dst, sa).wait()

    def phase1(g, b, acc):
        auxv, ia, ib, cf = auxs[b], ias[b], ibs[b], cfs[b]
        base = g * CHUNK
        wait_aux(g, b)
        for g2 in range(NG):
            s = g2 * 16
            j, sj = divmod(s, 128)
            c0 = auxv[pl.ds(s, 16)]
            c1 = auxv[pl.ds(CHUNK + s, 16)]
            p = base + s + iot
            q0 = lax.div(p, D).astype(jnp.float32)
            q1 = lax.rem(p, D).astype(jnp.float32)
            f0, ra, rb, d0 = _axis_terms(q0, c0, 1.0)
            f1, cca, ccb, d1 = _axis_terms(q1, c1, 1.0)
            idx = nR + f0 * D + f1
            ia[j, pl.ds(sj, 16)] = idx
            ia[NSUB + j, pl.ds(sj, 16)] = idx + D
            cf[0, pl.ds(s, 16)] = ra
            cf[1, pl.ds(s, 16)] = rb
            cf[2, pl.ds(s, 16)] = cca
            cf[3, pl.ds(s, 16)] = ccb
            acc = acc + d0 + d1
            f0, ra, rb, d0 = _axis_terms(q0, c0, -1.0)
            f1, cca, ccb, d1 = _axis_terms(q1, c1, -1.0)
            idx = nR + f0 * D + f1
            ib[j, pl.ds(sj, 16)] = idx
            ib[NSUB + j, pl.ds(sj, 16)] = idx + D
            cf[4, pl.ds(s, 16)] = ra
            cf[5, pl.ds(s, 16)] = rb
            cf[6, pl.ds(s, 16)] = cca
            cf[7, pl.ds(s, 16)] = ccb
            acc = acc + d0 + d1
        gb, sg = gs[b], semgs[b]
        for j in range(NSUB):
            pltpu.async_copy(t1.at[ia.at[j]],
                             gb.at[pl.ds(j * 128, 128)], sg)
            pltpu.async_copy(t1.at[ia.at[NSUB + j]],
                             gb.at[pl.ds(CHUNK + j * 128, 128)], sg)
            pltpu.async_copy(t2.at[ib.at[j]],
                             gb.at[pl.ds(2 * CHUNK + j * 128, 128)], sg)
            pltpu.async_copy(t2.at[ib.at[NSUB + j]],
                             gb.at[pl.ds(3 * CHUNK + j * 128, 128)], sg)
        return acc

    def wait_gathers(b):
        gb, sg = gs[b], semgs[b]
        for j in range(NSUB):
            for r in range(4):
                pltpu.make_async_copy(
                    t1.at[ias[b].at[j]],
                    gb.at[pl.ds(r * CHUNK + j * 128, 128)], sg).wait()

    def phase2(g, b):
        auxv, cf, gb, outv = auxs[b], cfs[b], gs[b], outvs[b]
        base = g * CHUNK
        @pl.when(g >= 2)
        def _():
            for ch in range(3):
                pltpu.make_async_copy(
                    outv.at[pl.ds(ch * CHUNK, CHUNK)],
                    out.at[pl.ds(ch * CHUNK, CHUNK)], semos[b]).wait()
        for g2 in range(NG):
            s = g2 * 16
            rows = s + iot
            m1 = auxv[pl.ds(2 * CHUNK + s, 16)]
            m2 = auxv[pl.ds(3 * CHUNK + s, 16)]
            ra1 = cf[0, pl.ds(s, 16)]
            rb1 = cf[1, pl.ds(s, 16)]
            ca1 = cf[2, pl.ds(s, 16)]
            cb1 = cf[3, pl.ds(s, 16)]
            ra2 = cf[4, pl.ds(s, 16)]
            rb2 = cf[5, pl.ds(s, 16)]
            ca2 = cf[6, pl.ds(s, 16)]
            cb2 = cf[7, pl.ds(s, 16)]
            for ch in range(3):
                c_lo = jnp.full((16,), ch, jnp.int32)
                c_hi = jnp.full((16,), ch + 3, jnp.int32)
                r1f = rows
                r1c = rows + CHUNK
                r2f = rows + 2 * CHUNK
                r2c = rows + 3 * CHUNK
                v1 = (ra1 * (ca1 * plsc.load_gather(gb, [r1f, c_lo])
                             + cb1 * plsc.load_gather(gb, [r1f, c_hi]))
                      + rb1 * (ca1 * plsc.load_gather(gb, [r1c, c_lo])
                               + cb1 * plsc.load_gather(gb, [r1c, c_hi])))
                v2 = (ra2 * (ca2 * plsc.load_gather(gb, [r2f, c_lo])
                             + cb2 * plsc.load_gather(gb, [r2f, c_hi]))
                      + rb2 * (ca2 * plsc.load_gather(gb, [r2c, c_lo])
                               + cb2 * plsc.load_gather(gb, [r2c, c_hi])))
                outv[pl.ds(ch * CHUNK + s, 16)] = v1 * m1 + v2 * m2
        for ch in range(3):
            pltpu.async_copy(
                outv.at[pl.ds(ch * CHUNK, CHUNK)],
                out.at[pl.ds((n * 3 + ch) * R + base, CHUNK)], semos[b])

    stgs = (stg0, stg1, stg2)

    def build_table(imf, tbuf):
        def blk_body(blk, carry):
            r0 = blk * BLOCK
            copies = [
                (imf.at[pl.ds((n * 3 + ch) * R + r0, BLOCK)],
                 stgs[ch].at[pl.ds(0, BLOCK)]) for ch in range(3)]
            for src, dst in copies:
                pltpu.async_copy(src, dst, semb)
            for src, dst in copies:
                pltpu.make_async_copy(src, dst, semb).wait()
            for ch in range(3):
                stg = stgs[ch]
                c_lo = jnp.full((16,), ch, jnp.int32)
                c_hi = jnp.full((16,), ch + 3, jnp.int32)

                def grp_body(j, carry2):
                    for g2 in range(8):
                        s = j * 128 + g2 * 16
                        rows = s + iot
                        plsc.store_scatter(ob, [rows, c_lo],
                                           stg[pl.ds(s, 16)])
                        plsc.store_scatter(ob, [rows, c_hi],
                                           stg[pl.ds(s + 1, 16)])
                    return carry2
                lax.fori_loop(0, BLOCK // 128, grp_body, 0)
            pltpu.sync_copy(ob, tbuf.at[pl.ds(nR + r0, BLOCK)])
            return carry
        lax.fori_loop(0, NBLK, blk_body, 0)

    build_table(im1f, t1)
    build_table(im2f, t2)

    fire_aux(0, 0)
    acc0 = phase1(0, 0, jnp.zeros((16,), jnp.float32))
    fire_aux(1, 1)

    def pair(k, acc):
        for b in range(2):
            g = 2 * k + b
            acc = lax.cond(
                g + 1 < NCHUNK,
                lambda a: phase1(g + 1, 1 - b, a),
                lambda a: a,
                acc)
            wait_gathers(b)
            phase2(g, b)

            @pl.when(g + 2 < NCHUNK)
            def _():
                fire_aux(g + 2, b)
        return acc

    acc = lax.fori_loop(0, NCHUNK // 2, pair, acc0)
    for b in range(2):
        for ch in range(3):
            pltpu.make_async_copy(
                outvs[b].at[pl.ds(ch * CHUNK, CHUNK)],
                out.at[pl.ds(ch * CHUNK, CHUNK)], semos[b]).wait()
    accs[...] = acc
    pltpu.sync_copy(accs, lossv.at[pl.ds(wid * 16, 16)])


@jax.jit
def _run(im1f, im2f, cfl, m1f, m2f):
    mesh = plsc.VectorSubcoreMesh(core_axis_name="c", subcore_axis_name="s")
    f = pl.kernel(
        _sc_body,
        out_type=[
            jax.ShapeDtypeStruct((N * 3 * R,), jnp.float32),
            jax.ShapeDtypeStruct((N * 16,), jnp.float32),
            jax.ShapeDtypeStruct((N * R, 8), jnp.float32),
            jax.ShapeDtypeStruct((N * R, 8), jnp.float32),
        ],
        mesh=mesh,
        compiler_params=pltpu.CompilerParams(
            needs_layout_passes=False, use_tc_tiling_on_sc=False),
        scratch_types=[
            pltpu.VMEM((4 * CHUNK,), jnp.float32),
            pltpu.VMEM((4 * CHUNK,), jnp.float32),
            pltpu.VMEM((2 * NSUB, 128), jnp.int32),
            pltpu.VMEM((2 * NSUB, 128), jnp.int32),
            pltpu.VMEM((2 * NSUB, 128), jnp.int32),
            pltpu.VMEM((2 * NSUB, 128), jnp.int32),
            pltpu.VMEM((8, CHUNK), jnp.float32),
            pltpu.VMEM((8, CHUNK), jnp.float32),
            pltpu.VMEM((4 * CHUNK, 8), jnp.float32),
            pltpu.VMEM((4 * CHUNK, 8), jnp.float32),
            pltpu.VMEM((3 * CHUNK,), jnp.float32),
            pltpu.VMEM((3 * CHUNK,), jnp.float32),
            pltpu.VMEM((16,), jnp.float32),
            pltpu.VMEM((BLOCK + 16,), jnp.float32),
            pltpu.VMEM((BLOCK + 16,), jnp.float32),
            pltpu.VMEM((BLOCK + 16,), jnp.float32),
            pltpu.VMEM((BLOCK, 8), jnp.float32),
            pltpu.SemaphoreType.DMA,
            pltpu.SemaphoreType.DMA,
            pltpu.SemaphoreType.DMA,
            pltpu.SemaphoreType.DMA,
            pltpu.SemaphoreType.DMA,
            pltpu.SemaphoreType.DMA,
            pltpu.SemaphoreType.DMA,
        ],
    )
    return f(im1f, im2f, cfl, m1f, m2f)


def kernel(im1, im2, C, M1, M2):
    out_flat, loss_part, _, _ = _run(
        im1.reshape(N * 3 * R), im2.reshape(N * 3 * R),
        C.reshape(N * 2 * R), M1.reshape(N * R), M2.reshape(N * R))
    out = out_flat.reshape(N, 3, D, D)
    loss = jnp.sum(loss_part) * (0.01 / (N * 2.0 * R * D * D))
    return out, loss


# speedup vs baseline: 3.0626x; 1.0729x over previous
import jax
import jax.numpy as jnp
from jax import lax
from jax.experimental import pallas as pl
from jax.experimental.pallas import tpu as pltpu
from jax.experimental.pallas import tpu_sc as plsc

D = 224
N = 32
R = D * D
CHUNK = 256
NSUB = CHUNK // 128
NCHUNK = R // CHUNK
NG = CHUNK // 16

_LO = 0.001
_HI = D - 1.001


def _axis_terms(qo, c_chunk, sign):
    q = qo + sign * c_chunk
    qc = jnp.minimum(jnp.maximum(q, _LO), _HI)
    fi = qc.astype(jnp.int32)
    ff = fi.astype(jnp.float32)
    frac_pos = qc > ff
    cf = ff + jnp.where(frac_pos, 1.0, 0.0)
    wf = 1.0 - (qc - ff)
    wc = 1.0 - (cf - qc)
    ca = wf + jnp.where(frac_pos, 0.0, wc)
    cb = jnp.where(frac_pos, wc, 0.0)
    d = q - qc
    return fi, ca, cb, d * d


BLOCK = 16 * D
NBLK = R // BLOCK


def _sc_body(im1f, im2f, cfl, m1f, m2f, out, lossv, t1, t2,
             auxv0, auxv1, ia0, ia1, ib0, ib1,
             cf0, cf1, g0, g1, outv0, outv1, accs,
             stg0, stg1, stg2, stg3, stg4, stg5, ob0, ob1,
             semg0, semg1, semo0, semo1, sema0, sema1,
             sembl0, sembl1, sembs0, sembs1):
    wid = lax.axis_index("s") * 2 + lax.axis_index("c")
    n = wid
    iot = lax.iota(jnp.int32, 16)
    nR = n * R
    auxs = (auxv0, auxv1)
    ias = (ia0, ia1)
    ibs = (ib0, ib1)
    cfs = (cf0, cf1)
    gs = (g0, g1)
    outvs = (outv0, outv1)
    semgs = (semg0, semg1)
    semos = (semo0, semo1)
    semas = (sema0, sema1)

    def aux_copies(g, b):
        auxv, sa = auxs[b], semas[b]
        base = g * CHUNK
        return (
            (cfl.at[pl.ds(n * 2 * R + base, CHUNK)],
             auxv.at[pl.ds(0, CHUNK)], sa),
            (cfl.at[pl.ds(n * 2 * R + R + base, CHUNK)],
             auxv.at[pl.ds(CHUNK, CHUNK)], sa),
            (m1f.at[pl.ds(n * R + base, CHUNK)],
             auxv.at[pl.ds(2 * CHUNK, CHUNK)], sa),
            (m2f.at[pl.ds(n * R + base, CHUNK)],
             auxv.at[pl.ds(3 * CHUNK, CHUNK)], sa),
        )

    def fire_aux(g, b):
        for src, dst, sa in aux_copies(g, b):
            pltpu.async_copy(src, dst, sa)

    def wait_aux(g, b):
        for src, dst, sa in aux_copies(g, b):
            pltpu.make_async_copy(src, dst, sa).wait()

    def phase1(g, b, acc):
        auxv, ia, ib, cf = auxs[b], ias[b], ibs[b], cfs[b]
        base = g * CHUNK
        wait_aux(g, b)
        for g2 in range(NG):
            s = g2 * 16
            j, sj = divmod(s, 128)
            c0 = auxv[pl.ds(s, 16)]
            c1 = auxv[pl.ds(CHUNK + s, 16)]
            p = base + s + iot
            q0 = lax.div(p, D).astype(jnp.float32)
            q1 = lax.rem(p, D).astype(jnp.float32)
            f0, ra, rb, d0 = _axis_terms(q0, c0, 1.0)
            f1, cca, ccb, d1 = _axis_terms(q1, c1, 1.0)
            idx = nR + f0 * D + f1
            ia[j, pl.ds(sj, 16)] = idx
            ia[NSUB + j, pl.ds(sj, 16)] = idx + D
            cf[0, pl.ds(s, 16)] = ra
            cf[1, pl.ds(s, 16)] = rb
            cf[2, pl.ds(s, 16)] = cca
            cf[3, pl.ds(s, 16)] = ccb
            acc = acc + d0 + d1
            f0, ra, rb, d0 = _axis_terms(q0, c0, -1.0)
            f1, cca, ccb, d1 = _axis_terms(q1, c1, -1.0)
            idx = nR + f0 * D + f1
            ib[j, pl.ds(sj, 16)] = idx
            ib[NSUB + j, pl.ds(sj, 16)] = idx + D
            cf[4, pl.ds(s, 16)] = ra
            cf[5, pl.ds(s, 16)] = rb
            cf[6, pl.ds(s, 16)] = cca
            cf[7, pl.ds(s, 16)] = ccb
            acc = acc + d0 + d1
        gb, sg = gs[b], semgs[b]
        for j in range(NSUB):
            pltpu.async_copy(t1.at[ia.at[j]],
                             gb.at[pl.ds(j * 128, 128)], sg)
            pltpu.async_copy(t1.at[ia.at[NSUB + j]],
                             gb.at[pl.ds(CHUNK + j * 128, 128)], sg)
            pltpu.async_copy(t2.at[ib.at[j]],
                             gb.at[pl.ds(2 * CHUNK + j * 128, 128)], sg)
            pltpu.async_copy(t2.at[ib.at[NSUB + j]],
                             gb.at[pl.ds(3 * CHUNK + j * 128, 128)], sg)
        return acc

    def wait_gathers(b):
        gb, sg = gs[b], semgs[b]
        for j in range(NSUB):
            for r in range(4):
                pltpu.make_async_copy(
                    t1.at[ias[b].at[j]],
                    gb.at[pl.ds(r * CHUNK + j * 128, 128)], sg).wait()

    def phase2(g, b):
        auxv, cf, gb, outv = auxs[b], cfs[b], gs[b], outvs[b]
        base = g * CHUNK
        @pl.when(g >= 2)
        def _():
            for ch in range(3):
                pltpu.make_async_copy(
                    outv.at[pl.ds(ch * CHUNK, CHUNK)],
                    out.at[pl.ds(ch * CHUNK, CHUNK)], semos[b]).wait()
        for g2 in range(NG):
            s = g2 * 16
            rows = s + iot
            m1 = auxv[pl.ds(2 * CHUNK + s, 16)]
            m2 = auxv[pl.ds(3 * CHUNK + s, 16)]
            ra1 = cf[0, pl.ds(s, 16)]
            rb1 = cf[1, pl.ds(s, 16)]
            ca1 = cf[2, pl.ds(s, 16)]
            cb1 = cf[3, pl.ds(s, 16)]
            ra2 = cf[4, pl.ds(s, 16)]
            rb2 = cf[5, pl.ds(s, 16)]
            ca2 = cf[6, pl.ds(s, 16)]
            cb2 = cf[7, pl.ds(s, 16)]
            for ch in range(3):
                c_lo = jnp.full((16,), ch, jnp.int32)
                c_hi = jnp.full((16,), ch + 3, jnp.int32)
                r1f = rows
                r1c = rows + CHUNK
                r2f = rows + 2 * CHUNK
                r2c = rows + 3 * CHUNK
                v1 = (ra1 * (ca1 * plsc.load_gather(gb, [r1f, c_lo])
                             + cb1 * plsc.load_gather(gb, [r1f, c_hi]))
                      + rb1 * (ca1 * plsc.load_gather(gb, [r1c, c_lo])
                               + cb1 * plsc.load_gather(gb, [r1c, c_hi])))
                v2 = (ra2 * (ca2 * plsc.load_gather(gb, [r2f, c_lo])
                             + cb2 * plsc.load_gather(gb, [r2f, c_hi]))
                      + rb2 * (ca2 * plsc.load_gather(gb, [r2c, c_lo])
                               + cb2 * plsc.load_gather(gb, [r2c, c_hi])))
                outv[pl.ds(ch * CHUNK + s, 16)] = v1 * m1 + v2 * m2
        for ch in range(3):
            pltpu.async_copy(
                outv.at[pl.ds(ch * CHUNK, CHUNK)],
                out.at[pl.ds((n * 3 + ch) * R + base, CHUNK)], semos[b])

    stgsets = ((stg0, stg1, stg2), (stg3, stg4, stg5))
    obs = (ob0, ob1)
    sembls = (sembl0, sembl1)
    sembss = (sembs0, sembs1)

    def build_table(imf, tbuf):
        def stage_copies(blk, sb):
            return [
                (imf.at[pl.ds((n * 3 + ch) * R + blk * BLOCK, BLOCK)],
                 stgsets[sb][ch].at[pl.ds(0, BLOCK)]) for ch in range(3)]

        def fire_stage(blk, sb):
            for src, dst in stage_copies(blk, sb):
                pltpu.async_copy(src, dst, sembls[sb])

        def wait_stage(blk, sb):
            for src, dst in stage_copies(blk, sb):
                pltpu.make_async_copy(src, dst, sembls[sb]).wait()

        fire_stage(0, 0)

        def blk_pair(k, carry):
            for sb in range(2):
                blk = 2 * k + sb
                wait_stage(blk, sb)

                @pl.when(blk + 1 < NBLK)
                def _():
                    fire_stage(blk + 1, 1 - sb)

                @pl.when(blk >= 2)
                def _():
                    pltpu.make_async_copy(
                        obs[sb], tbuf.at[pl.ds(nR, BLOCK)],
                        sembss[sb]).wait()
                for ch in range(3):
                    stg = stgsets[sb][ch]
                    ob = obs[sb]
                    c_lo = jnp.full((16,), ch, jnp.int32)
                    c_hi = jnp.full((16,), ch + 3, jnp.int32)

                    def grp_body(j, carry2, stg=stg, ob=ob,
                                 c_lo=c_lo, c_hi=c_hi):
                        for g2 in range(8):
                            s = j * 128 + g2 * 16
                            rows = s + iot
                            plsc.store_scatter(ob, [rows, c_lo],
                                               stg[pl.ds(s, 16)])
                            plsc.store_scatter(ob, [rows, c_hi],
                                               stg[pl.ds(s + 1, 16)])
                        return carry2
                    lax.fori_loop(0, BLOCK // 128, grp_body, 0)
                pltpu.async_copy(obs[sb],
                                 tbuf.at[pl.ds(nR + blk * BLOCK, BLOCK)],
                                 sembss[sb])
            return carry
        lax.fori_loop(0, NBLK // 2, blk_pair, 0)
        for sb in range(2):
            pltpu.make_async_copy(
                obs[sb], tbuf.at[pl.ds(nR, BLOCK)], sembss[sb]).wait()

    build_table(im1f, t1)
    build_table(im2f, t2)

    fire_aux(0, 0)
    acc0 = phase1(0, 0, jnp.zeros((16,), jnp.float32))
    fire_aux(1, 1)

    def pair(k, acc):
        for b in range(2):
            g = 2 * k + b
            acc = lax.cond(
                g + 1 < NCHUNK,
                lambda a: phase1(g + 1, 1 - b, a),
                lambda a: a,
                acc)
            wait_gathers(b)
            phase2(g, b)

            @pl.when(g + 2 < NCHUNK)
            def _():
                fire_aux(g + 2, b)
        return acc

    acc = lax.fori_loop(0, NCHUNK // 2, pair, acc0)
    for b in range(2):
        for ch in range(3):
            pltpu.make_async_copy(
                outvs[b].at[pl.ds(ch * CHUNK, CHUNK)],
                out.at[pl.ds(ch * CHUNK, CHUNK)], semos[b]).wait()
    accs[...] = acc
    pltpu.sync_copy(accs, lossv.at[pl.ds(wid * 16, 16)])


@jax.jit
def _run(im1f, im2f, cfl, m1f, m2f):
    mesh = plsc.VectorSubcoreMesh(core_axis_name="c", subcore_axis_name="s")
    f = pl.kernel(
        _sc_body,
        out_type=[
            jax.ShapeDtypeStruct((N * 3 * R,), jnp.float32),
            jax.ShapeDtypeStruct((N * 16,), jnp.float32),
            jax.ShapeDtypeStruct((N * R, 8), jnp.float32),
            jax.ShapeDtypeStruct((N * R, 8), jnp.float32),
        ],
        mesh=mesh,
        compiler_params=pltpu.CompilerParams(
            needs_layout_passes=False, use_tc_tiling_on_sc=False),
        scratch_types=[
            pltpu.VMEM((4 * CHUNK,), jnp.float32),
            pltpu.VMEM((4 * CHUNK,), jnp.float32),
            pltpu.VMEM((2 * NSUB, 128), jnp.int32),
            pltpu.VMEM((2 * NSUB, 128), jnp.int32),
            pltpu.VMEM((2 * NSUB, 128), jnp.int32),
            pltpu.VMEM((2 * NSUB, 128), jnp.int32),
            pltpu.VMEM((8, CHUNK), jnp.float32),
            pltpu.VMEM((8, CHUNK), jnp.float32),
            pltpu.VMEM((4 * CHUNK, 8), jnp.float32),
            pltpu.VMEM((4 * CHUNK, 8), jnp.float32),
            pltpu.VMEM((3 * CHUNK,), jnp.float32),
            pltpu.VMEM((3 * CHUNK,), jnp.float32),
            pltpu.VMEM((16,), jnp.float32),
            pltpu.VMEM((BLOCK + 16,), jnp.float32),
            pltpu.VMEM((BLOCK + 16,), jnp.float32),
            pltpu.VMEM((BLOCK + 16,), jnp.float32),
            pltpu.VMEM((BLOCK + 16,), jnp.float32),
            pltpu.VMEM((BLOCK + 16,), jnp.float32),
            pltpu.VMEM((BLOCK + 16,), jnp.float32),
            pltpu.VMEM((BLOCK, 8), jnp.float32),
            pltpu.VMEM((BLOCK, 8), jnp.float32),
            pltpu.SemaphoreType.DMA,
            pltpu.SemaphoreType.DMA,
            pltpu.SemaphoreType.DMA,
            pltpu.SemaphoreType.DMA,
            pltpu.SemaphoreType.DMA,
            pltpu.SemaphoreType.DMA,
            pltpu.SemaphoreType.DMA,
            pltpu.SemaphoreType.DMA,
            pltpu.SemaphoreType.DMA,
            pltpu.SemaphoreType.DMA,
        ],
    )
    return f(im1f, im2f, cfl, m1f, m2f)


def kernel(im1, im2, C, M1, M2):
    out_flat, loss_part, _, _ = _run(
        im1.reshape(N * 3 * R), im2.reshape(N * 3 * R),
        C.reshape(N * 2 * R), M1.reshape(N * R), M2.reshape(N * R))
    out = out_flat.reshape(N, 3, D, D)
    loss = jnp.sum(loss_part) * (0.01 / (N * 2.0 * R * D * D))
    return out, loss


# speedup vs baseline: 5.0416x; 1.6462x over previous
import jax
import jax.numpy as jnp
from jax import lax
from jax.experimental import pallas as pl
from jax.experimental.pallas import tpu as pltpu
from jax.experimental.pallas import tpu_sc as plsc

D = 224
N = 32
R = D * D
CHUNK = 512
NSUB = CHUNK // 128
NCHUNK = R // CHUNK
NG = CHUNK // 16

_LO = 0.001
_HI = D - 1.001


def _axis_terms(qo, c_chunk, sign):
    q = qo + sign * c_chunk
    qc = jnp.minimum(jnp.maximum(q, _LO), _HI)
    fi = qc.astype(jnp.int32)
    ff = fi.astype(jnp.float32)
    frac_pos = qc > ff
    cf = ff + jnp.where(frac_pos, 1.0, 0.0)
    wf = 1.0 - (qc - ff)
    wc = 1.0 - (cf - qc)
    ca = wf + jnp.where(frac_pos, 0.0, wc)
    cb = jnp.where(frac_pos, wc, 0.0)
    d = q - qc
    return fi, ca, cb, d * d


BLOCK = 8 * D
NBLK = R // BLOCK


def _sc_body(im1f, im2f, cfl, m1f, m2f, out, lossv, t1, t2,
             auxv0, auxv1, ia0, ia1, ib0, ib1,
             cf0, cf1, g0, g1, outv0, outv1, accs,
             stg0, stg1, stg2, stg3, stg4, stg5, ob0, ob1,
             semg0, semg1, semo0, semo1, sema0, sema1,
             sembl0, sembl1, sembs0, sembs1):
    wid = lax.axis_index("s") * 2 + lax.axis_index("c")
    n = wid
    iot = lax.iota(jnp.int32, 16)
    nR = n * R
    auxs = (auxv0, auxv1)
    ias = (ia0, ia1)
    ibs = (ib0, ib1)
    cfs = (cf0, cf1)
    gs = (g0, g1)
    outvs = (outv0, outv1)
    semgs = (semg0, semg1)
    semos = (semo0, semo1)
    semas = (sema0, sema1)

    def aux_copies(g, b):
        auxv, sa = auxs[b], semas[b]
        base = g * CHUNK
        return (
            (cfl.at[pl.ds(n * 2 * R + base, CHUNK)],
             auxv.at[pl.ds(0, CHUNK)], sa),
            (cfl.at[pl.ds(n * 2 * R + R + base, CHUNK)],
             auxv.at[pl.ds(CHUNK, CHUNK)], sa),
            (m1f.at[pl.ds(n * R + base, CHUNK)],
             auxv.at[pl.ds(2 * CHUNK, CHUNK)], sa),
            (m2f.at[pl.ds(n * R + base, CHUNK)],
             auxv.at[pl.ds(3 * CHUNK, CHUNK)], sa),
        )

    def fire_aux(g, b):
        for src, dst, sa in aux_copies(g, b):
            pltpu.async_copy(src, dst, sa)

    def wait_aux(g, b):
        for src, dst, sa in aux_copies(g, b):
            pltpu.make_async_copy(src, dst, sa).wait()

    def phase1(g, b, acc):
        auxv, ia, ib, cf = auxs[b], ias[b], ibs[b], cfs[b]
        base = g * CHUNK
        wait_aux(g, b)

        def p1_body(g2, a):
            s = g2 * 16
            jj = lax.div(g2, 8)
            sj = lax.rem(g2, 8) * 16
            c0 = auxv[pl.ds(s, 16)]
            c1 = auxv[pl.ds(CHUNK + s, 16)]
            m1 = auxv[pl.ds(2 * CHUNK + s, 16)]
            m2 = auxv[pl.ds(3 * CHUNK + s, 16)]
            p = base + s + iot
            q0 = lax.div(p, D).astype(jnp.float32)
            q1 = lax.rem(p, D).astype(jnp.float32)
            f0, ra, rb, d0 = _axis_terms(q0, c0, 1.0)
            f1, cca, ccb, d1 = _axis_terms(q1, c1, 1.0)
            idx = nR + f0 * D + f1
            ia[jj, pl.ds(sj, 16)] = idx
            ia[NSUB + jj, pl.ds(sj, 16)] = idx + D
            mra = m1 * ra
            mrb = m1 * rb
            cf[0, pl.ds(s, 16)] = mra * cca
            cf[1, pl.ds(s, 16)] = mra * ccb
            cf[2, pl.ds(s, 16)] = mrb * cca
            cf[3, pl.ds(s, 16)] = mrb * ccb
            a = a + d0 + d1
            f0, ra, rb, d0 = _axis_terms(q0, c0, -1.0)
            f1, cca, ccb, d1 = _axis_terms(q1, c1, -1.0)
            idx = nR + f0 * D + f1
            ib[jj, pl.ds(sj, 16)] = idx
            ib[NSUB + jj, pl.ds(sj, 16)] = idx + D
            mra = m2 * ra
            mrb = m2 * rb
            cf[4, pl.ds(s, 16)] = mra * cca
            cf[5, pl.ds(s, 16)] = mra * ccb
            cf[6, pl.ds(s, 16)] = mrb * cca
            cf[7, pl.ds(s, 16)] = mrb * ccb
            return a + d0 + d1

        acc = lax.fori_loop(0, NG, p1_body, acc)
        gb, sg = gs[b], semgs[b]
        for j in range(NSUB):
            pltpu.async_copy(t1.at[ia.at[j]],
                             gb.at[pl.ds(j * 128, 128)], sg)
            pltpu.async_copy(t1.at[ia.at[NSUB + j]],
                             gb.at[pl.ds(CHUNK + j * 128, 128)], sg)
            pltpu.async_copy(t2.at[ib.at[j]],
                             gb.at[pl.ds(2 * CHUNK + j * 128, 128)], sg)
            pltpu.async_copy(t2.at[ib.at[NSUB + j]],
                             gb.at[pl.ds(3 * CHUNK + j * 128, 128)], sg)
        return acc

    def wait_gathers(b):
        gb, sg = gs[b], semgs[b]
        for j in range(NSUB):
            for r in range(4):
                pltpu.make_async_copy(
                    t1.at[ias[b].at[j]],
                    gb.at[pl.ds(r * CHUNK + j * 128, 128)], sg).wait()

    def phase2(g, b):
        auxv, cf, gb, outv = auxs[b], cfs[b], gs[b], outvs[b]
        base = g * CHUNK
        @pl.when(g >= 2)
        def _():
            for ch in range(3):
                pltpu.make_async_copy(
                    outv.at[pl.ds(ch * CHUNK, CHUNK)],
                    out.at[pl.ds(ch * CHUNK, CHUNK)], semos[b]).wait()
        def p2_body(g2, carry):
            s = g2 * 16
            rows = s + iot
            w11 = cf[0, pl.ds(s, 16)]
            w12 = cf[1, pl.ds(s, 16)]
            w21 = cf[2, pl.ds(s, 16)]
            w22 = cf[3, pl.ds(s, 16)]
            w31 = cf[4, pl.ds(s, 16)]
            w32 = cf[5, pl.ds(s, 16)]
            w41 = cf[6, pl.ds(s, 16)]
            w42 = cf[7, pl.ds(s, 16)]
            r1f = rows
            r1c = rows + CHUNK
            r2f = rows + 2 * CHUNK
            r2c = rows + 3 * CHUNK
            for ch in range(3):
                c_lo = jnp.full((16,), ch, jnp.int32)
                c_hi = jnp.full((16,), ch + 3, jnp.int32)
                v = (w11 * plsc.load_gather(gb, [r1f, c_lo])
                     + w12 * plsc.load_gather(gb, [r1f, c_hi])
                     + w21 * plsc.load_gather(gb, [r1c, c_lo])
                     + w22 * plsc.load_gather(gb, [r1c, c_hi])
                     + w31 * plsc.load_gather(gb, [r2f, c_lo])
                     + w32 * plsc.load_gather(gb, [r2f, c_hi])
                     + w41 * plsc.load_gather(gb, [r2c, c_lo])
                     + w42 * plsc.load_gather(gb, [r2c, c_hi]))
                outv[pl.ds(ch * CHUNK + s, 16)] = v
            return carry

        lax.fori_loop(0, NG, p2_body, 0)
        for ch in range(3):
            pltpu.async_copy(
                outv.at[pl.ds(ch * CHUNK, CHUNK)],
                out.at[pl.ds((n * 3 + ch) * R + base, CHUNK)], semos[b])

    stgsets = ((stg0, stg1, stg2), (stg3, stg4, stg5))
    obs = (ob0, ob1)
    sembls = (sembl0, sembl1)
    sembss = (sembs0, sembs1)

    def build_table(imf, tbuf):
        def stage_copies(blk, sb):
            return [
                (imf.at[pl.ds((n * 3 + ch) * R + blk * BLOCK, BLOCK)],
                 stgsets[sb][ch].at[pl.ds(0, BLOCK)]) for ch in range(3)]

        def fire_stage(blk, sb):
            for src, dst in stage_copies(blk, sb):
                pltpu.async_copy(src, dst, sembls[sb])

        def wait_stage(blk, sb):
            for src, dst in stage_copies(blk, sb):
                pltpu.make_async_copy(src, dst, sembls[sb]).wait()

        fire_stage(0, 0)

        def blk_pair(k, carry):
            for sb in range(2):
                blk = 2 * k + sb
                wait_stage(blk, sb)

                @pl.when(blk + 1 < NBLK)
                def _():
                    fire_stage(blk + 1, 1 - sb)

                @pl.when(blk >= 2)
                def _():
                    pltpu.make_async_copy(
                        obs[sb], tbuf.at[pl.ds(nR, BLOCK)],
                        sembss[sb]).wait()
                for ch in range(3):
                    stg = stgsets[sb][ch]
                    ob = obs[sb]
                    c_lo = jnp.full((16,), ch, jnp.int32)
                    c_hi = jnp.full((16,), ch + 3, jnp.int32)

                    def grp_body(j, carry2, stg=stg, ob=ob,
                                 c_lo=c_lo, c_hi=c_hi):
                        for g2 in range(8):
                            s = j * 128 + g2 * 16
                            rows = s + iot
                            plsc.store_scatter(ob, [rows, c_lo],
                                               stg[pl.ds(s, 16)])
                            plsc.store_scatter(ob, [rows, c_hi],
                                               stg[pl.ds(s + 1, 16)])
                        return carry2
                    lax.fori_loop(0, BLOCK // 128, grp_body, 0)
                pltpu.async_copy(obs[sb],
                                 tbuf.at[pl.ds(nR + blk * BLOCK, BLOCK)],
                                 sembss[sb])
            return carry
        lax.fori_loop(0, NBLK // 2, blk_pair, 0)
        for sb in range(2):
            pltpu.make_async_copy(
                obs[sb], tbuf.at[pl.ds(nR, BLOCK)], sembss[sb]).wait()

    build_table(im1f, t1)
    build_table(im2f, t2)

    fire_aux(0, 0)
    acc0 = phase1(0, 0, jnp.zeros((16,), jnp.float32))
    fire_aux(1, 1)

    def pair(k, acc):
        for b in range(2):
            g = 2 * k + b
            acc = lax.cond(
                g + 1 < NCHUNK,
                lambda a: phase1(g + 1, 1 - b, a),
                lambda a: a,
                acc)
            wait_gathers(b)
            phase2(g, b)

            @pl.when(g + 2 < NCHUNK)
            def _():
                fire_aux(g + 2, b)
        return acc

    acc = lax.fori_loop(0, NCHUNK // 2, pair, acc0)
    for b in range(2):
        for ch in range(3):
            pltpu.make_async_copy(
                outvs[b].at[pl.ds(ch * CHUNK, CHUNK)],
                out.at[pl.ds(ch * CHUNK, CHUNK)], semos[b]).wait()
    accs[...] = acc
    pltpu.sync_copy(accs, lossv.at[pl.ds(wid * 16, 16)])


@jax.jit
def _run(im1f, im2f, cfl, m1f, m2f):
    mesh = plsc.VectorSubcoreMesh(core_axis_name="c", subcore_axis_name="s")
    f = pl.kernel(
        _sc_body,
        out_type=[
            jax.ShapeDtypeStruct((N * 3 * R,), jnp.float32),
            jax.ShapeDtypeStruct((N * 16,), jnp.float32),
            jax.ShapeDtypeStruct((N * R, 8), jnp.float32),
            jax.ShapeDtypeStruct((N * R, 8), jnp.float32),
        ],
        mesh=mesh,
        compiler_params=pltpu.CompilerParams(
            needs_layout_passes=False, use_tc_tiling_on_sc=False),
        scratch_types=[
            pltpu.VMEM((4 * CHUNK,), jnp.float32),
            pltpu.VMEM((4 * CHUNK,), jnp.float32),
            pltpu.VMEM((2 * NSUB, 128), jnp.int32),
            pltpu.VMEM((2 * NSUB, 128), jnp.int32),
            pltpu.VMEM((2 * NSUB, 128), jnp.int32),
            pltpu.VMEM((2 * NSUB, 128), jnp.int32),
            pltpu.VMEM((8, CHUNK), jnp.float32),
            pltpu.VMEM((8, CHUNK), jnp.float32),
            pltpu.VMEM((4 * CHUNK, 8), jnp.float32),
            pltpu.VMEM((4 * CHUNK, 8), jnp.float32),
            pltpu.VMEM((3 * CHUNK,), jnp.float32),
            pltpu.VMEM((3 * CHUNK,), jnp.float32),
            pltpu.VMEM((16,), jnp.float32),
            pltpu.VMEM((BLOCK + 16,), jnp.float32),
            pltpu.VMEM((BLOCK + 16,), jnp.float32),
            pltpu.VMEM((BLOCK + 16,), jnp.float32),
            pltpu.VMEM((BLOCK + 16,), jnp.float32),
            pltpu.VMEM((BLOCK + 16,), jnp.float32),
            pltpu.VMEM((BLOCK + 16,), jnp.float32),
            pltpu.VMEM((BLOCK, 8), jnp.float32),
            pltpu.VMEM((BLOCK, 8), jnp.float32),
            pltpu.SemaphoreType.DMA,
            pltpu.SemaphoreType.DMA,
            pltpu.SemaphoreType.DMA,
            pltpu.SemaphoreType.DMA,
            pltpu.SemaphoreType.DMA,
            pltpu.SemaphoreType.DMA,
            pltpu.SemaphoreType.DMA,
            pltpu.SemaphoreType.DMA,
            pltpu.SemaphoreType.DMA,
            pltpu.SemaphoreType.DMA,
        ],
    )
    return f(im1f, im2f, cfl, m1f, m2f)


def kernel(im1, im2, C, M1, M2):
    out_flat, loss_part, _, _ = _run(
        im1.reshape(N * 3 * R), im2.reshape(N * 3 * R),
        C.reshape(N * 2 * R), M1.reshape(N * R), M2.reshape(N * R))
    out = out_flat.reshape(N, 3, D, D)
    loss = jnp.sum(loss_part) * (0.01 / (N * 2.0 * R * D * D))
    return out, loss


# speedup vs baseline: 5.0428x; 1.0002x over previous
import jax
import jax.numpy as jnp
from jax import lax
from jax.experimental import pallas as pl
from jax.experimental.pallas import tpu as pltpu
from jax.experimental.pallas import tpu_sc as plsc

D = 224
N = 32
R = D * D
CHUNK = 512
NSUB = CHUNK // 128
NCHUNK = R // CHUNK
NG = CHUNK // 16

_LO = 0.001
_HI = D - 1.001


def _axis_terms(qo, c_chunk, sign):
    q = qo + sign * c_chunk
    qc = jnp.minimum(jnp.maximum(q, _LO), _HI)
    fi = qc.astype(jnp.int32)
    ff = fi.astype(jnp.float32)
    frac_pos = qc > ff
    cf = ff + jnp.where(frac_pos, 1.0, 0.0)
    wf = 1.0 - (qc - ff)
    wc = 1.0 - (cf - qc)
    ca = wf + jnp.where(frac_pos, 0.0, wc)
    cb = jnp.where(frac_pos, wc, 0.0)
    d = q - qc
    return fi, ca, cb, d * d


BLOCK = 8 * D
NBLK = R // BLOCK


def _sc_body(im1f, im2f, cfl, m1f, m2f, out, lossv, t1, t2,
             auxv0, auxv1, ia0, ia1, ib0, ib1,
             cf0, cf1, g0, g1, outv0, outv1, accs,
             stg0, stg1, stg2, stg3, stg4, stg5, ob0, ob1,
             semg0, semg1, semo0, semo1, sema0, sema1,
             sembl0, sembl1, sembs0, sembs1):
    wid = lax.axis_index("s") * 2 + lax.axis_index("c")
    n = wid
    iot = lax.iota(jnp.int32, 16)
    nR = n * R
    auxs = (auxv0, auxv1)
    ias = (ia0, ia1)
    ibs = (ib0, ib1)
    cfs = (cf0, cf1)
    gs = (g0, g1)
    outvs = (outv0, outv1)
    semgs = (semg0, semg1)
    semos = (semo0, semo1)
    semas = (sema0, sema1)

    def aux_copies(g, b):
        auxv, sa = auxs[b], semas[b]
        base = g * CHUNK
        return (
            (cfl.at[pl.ds(n * 2 * R + base, CHUNK)],
             auxv.at[pl.ds(0, CHUNK)], sa),
            (cfl.at[pl.ds(n * 2 * R + R + base, CHUNK)],
             auxv.at[pl.ds(CHUNK, CHUNK)], sa),
            (m1f.at[pl.ds(n * R + base, CHUNK)],
             auxv.at[pl.ds(2 * CHUNK, CHUNK)], sa),
            (m2f.at[pl.ds(n * R + base, CHUNK)],
             auxv.at[pl.ds(3 * CHUNK, CHUNK)], sa),
        )

    def fire_aux(g, b):
        for src, dst, sa in aux_copies(g, b):
            pltpu.async_copy(src, dst, sa)

    def wait_aux(g, b):
        for src, dst, sa in aux_copies(g, b):
            pltpu.make_async_copy(src, dst, sa).wait()

    def phase1(g, b, acc):
        auxv, ia, ib, cf = auxs[b], ias[b], ibs[b], cfs[b]
        base = g * CHUNK
        wait_aux(g, b)

        def p1_body(g2, a):
            s = g2 * 16
            jj = lax.div(g2, 8)
            sj = lax.rem(g2, 8) * 16
            c0 = auxv[pl.ds(s, 16)]
            c1 = auxv[pl.ds(CHUNK + s, 16)]
            m1 = auxv[pl.ds(2 * CHUNK + s, 16)]
            m2 = auxv[pl.ds(3 * CHUNK + s, 16)]
            p = base + s + iot
            q0 = lax.div(p, D).astype(jnp.float32)
            q1 = lax.rem(p, D).astype(jnp.float32)
            f0, ra, rb, d0 = _axis_terms(q0, c0, 1.0)
            f1, cca, ccb, d1 = _axis_terms(q1, c1, 1.0)
            idx = nR + f0 * D + f1
            ia[jj, pl.ds(sj, 16)] = idx
            ia[NSUB + jj, pl.ds(sj, 16)] = idx + D
            mra = m1 * ra
            mrb = m1 * rb
            cf[0, pl.ds(s, 16)] = mra * cca
            cf[1, pl.ds(s, 16)] = mra * ccb
            cf[2, pl.ds(s, 16)] = mrb * cca
            cf[3, pl.ds(s, 16)] = mrb * ccb
            a = a + d0 + d1
            f0, ra, rb, d0 = _axis_terms(q0, c0, -1.0)
            f1, cca, ccb, d1 = _axis_terms(q1, c1, -1.0)
            idx = nR + f0 * D + f1
            ib[jj, pl.ds(sj, 16)] = idx
            ib[NSUB + jj, pl.ds(sj, 16)] = idx + D
            mra = m2 * ra
            mrb = m2 * rb
            cf[4, pl.ds(s, 16)] = mra * cca
            cf[5, pl.ds(s, 16)] = mra * ccb
            cf[6, pl.ds(s, 16)] = mrb * cca
            cf[7, pl.ds(s, 16)] = mrb * ccb
            return a + d0 + d1

        acc = lax.fori_loop(0, NG, p1_body, acc)
        gb, sg = gs[b], semgs[b]
        for j in range(NSUB):
            pltpu.async_copy(t1.at[ia.at[j]],
                             gb.at[pl.ds(j * 128, 128)], sg)
            pltpu.async_copy(t1.at[ia.at[NSUB + j]],
                             gb.at[pl.ds(CHUNK + j * 128, 128)], sg)
            pltpu.async_copy(t2.at[ib.at[j]],
                             gb.at[pl.ds(2 * CHUNK + j * 128, 128)], sg)
            pltpu.async_copy(t2.at[ib.at[NSUB + j]],
                             gb.at[pl.ds(3 * CHUNK + j * 128, 128)], sg)
        return acc

    def wait_gathers(b):
        gb, sg = gs[b], semgs[b]
        for j in range(NSUB):
            for r in range(4):
                pltpu.make_async_copy(
                    t1.at[ias[b].at[j]],
                    gb.at[pl.ds(r * CHUNK + j * 128, 128)], sg).wait()

    def phase2(g, b):
        auxv, cf, gb, outv = auxs[b], cfs[b], gs[b], outvs[b]
        base = g * CHUNK
        @pl.when(g >= 2)
        def _():
            for ch in range(3):
                pltpu.make_async_copy(
                    outv.at[pl.ds(ch * CHUNK, CHUNK)],
                    out.at[pl.ds(ch * CHUNK, CHUNK)], semos[b]).wait()
        def p2_body(g2, carry):
            s = g2 * 16
            rows = s + iot
            w11 = cf[0, pl.ds(s, 16)]
            w12 = cf[1, pl.ds(s, 16)]
            w21 = cf[2, pl.ds(s, 16)]
            w22 = cf[3, pl.ds(s, 16)]
            w31 = cf[4, pl.ds(s, 16)]
            w32 = cf[5, pl.ds(s, 16)]
            w41 = cf[6, pl.ds(s, 16)]
            w42 = cf[7, pl.ds(s, 16)]
            r1f = rows
            r1c = rows + CHUNK
            r2f = rows + 2 * CHUNK
            r2c = rows + 3 * CHUNK
            for ch in range(3):
                c_lo = jnp.full((16,), ch, jnp.int32)
                c_hi = jnp.full((16,), ch + 3, jnp.int32)
                v = (w11 * plsc.load_gather(gb, [r1f, c_lo])
                     + w12 * plsc.load_gather(gb, [r1f, c_hi])
                     + w21 * plsc.load_gather(gb, [r1c, c_lo])
                     + w22 * plsc.load_gather(gb, [r1c, c_hi])
                     + w31 * plsc.load_gather(gb, [r2f, c_lo])
                     + w32 * plsc.load_gather(gb, [r2f, c_hi])
                     + w41 * plsc.load_gather(gb, [r2c, c_lo])
                     + w42 * plsc.load_gather(gb, [r2c, c_hi]))
                outv[pl.ds(ch * CHUNK + s, 16)] = v
            return carry

        lax.fori_loop(0, NG, p2_body, 0)
        for ch in range(3):
            pltpu.async_copy(
                outv.at[pl.ds(ch * CHUNK, CHUNK)],
                out.at[pl.ds((n * 3 + ch) * R + base, CHUNK)], semos[b])

    stgsets = ((stg0, stg1, stg2), (stg3, stg4, stg5))
    obs = (ob0, ob1)
    sembls = (sembl0, sembl1)
    sembss = (sembs0, sembs1)

    def build_table(imf, tbuf):
        def stage_copies(blk, sb):
            return [
                (imf.at[pl.ds((n * 3 + ch) * R + blk * BLOCK, BLOCK)],
                 stgsets[sb][ch].at[pl.ds(0, BLOCK)]) for ch in range(3)]

        def fire_stage(blk, sb):
            for src, dst in stage_copies(blk, sb):
                pltpu.async_copy(src, dst, sembls[sb])

        def wait_stage(blk, sb):
            for src, dst in stage_copies(blk, sb):
                pltpu.make_async_copy(src, dst, sembls[sb]).wait()

        fire_stage(0, 0)

        def blk_pair(k, carry):
            for sb in range(2):
                blk = 2 * k + sb
                wait_stage(blk, sb)

                @pl.when(blk + 1 < NBLK)
                def _():
                    fire_stage(blk + 1, 1 - sb)

                @pl.when(blk >= 2)
                def _():
                    pltpu.make_async_copy(
                        obs[sb], tbuf.at[pl.ds(nR, BLOCK)],
                        sembss[sb]).wait()
                for ch in range(3):
                    stg = stgsets[sb][ch]
                    ob = obs[sb]
                    c_lo = jnp.full((16,), ch, jnp.int32)
                    c_hi = jnp.full((16,), ch + 3, jnp.int32)

                    def grp_body(j, carry2, stg=stg, ob=ob,
                                 c_lo=c_lo, c_hi=c_hi):
                        for g2 in range(8):
                            s = j * 128 + g2 * 16
                            rows = s + iot
                            plsc.store_scatter(ob, [rows, c_lo],
                                               stg[pl.ds(s, 16)])
                            plsc.store_scatter(ob, [rows, c_hi],
                                               stg[pl.ds(s + 1, 16)])
                        return carry2
                    lax.fori_loop(0, BLOCK // 128, grp_body, 0)
                pltpu.async_copy(obs[sb],
                                 tbuf.at[pl.ds(nR + blk * BLOCK, BLOCK)],
                                 sembss[sb])
            return carry
        lax.fori_loop(0, NBLK // 2, blk_pair, 0)
        for sb in range(2):
            pltpu.make_async_copy(
                obs[sb], tbuf.at[pl.ds(nR, BLOCK)], sembss[sb]).wait()

    fire_aux(0, 0)
    fire_aux(1, 1)
    build_table(im1f, t1)
    build_table(im2f, t2)

    acc0 = phase1(0, 0, jnp.zeros((16,), jnp.float32))

    def pair(k, acc):
        for b in range(2):
            g = 2 * k + b
            acc = lax.cond(
                g + 1 < NCHUNK,
                lambda a: phase1(g + 1, 1 - b, a),
                lambda a: a,
                acc)
            wait_gathers(b)
            phase2(g, b)

            @pl.when(g + 2 < NCHUNK)
            def _():
                fire_aux(g + 2, b)
        return acc

    acc = lax.fori_loop(0, NCHUNK // 2, pair, acc0)
    for b in range(2):
        for ch in range(3):
            pltpu.make_async_copy(
                outvs[b].at[pl.ds(ch * CHUNK, CHUNK)],
                out.at[pl.ds(ch * CHUNK, CHUNK)], semos[b]).wait()
    accs[...] = acc
    pltpu.sync_copy(accs, lossv.at[pl.ds(wid * 16, 16)])


@jax.jit
def _run(im1f, im2f, cfl, m1f, m2f):
    mesh = plsc.VectorSubcoreMesh(core_axis_name="c", subcore_axis_name="s")
    f = pl.kernel(
        _sc_body,
        out_type=[
            jax.ShapeDtypeStruct((N * 3 * R,), jnp.float32),
            jax.ShapeDtypeStruct((N * 16,), jnp.float32),
            jax.ShapeDtypeStruct((N * R, 8), jnp.float32),
            jax.ShapeDtypeStruct((N * R, 8), jnp.float32),
        ],
        mesh=mesh,
        compiler_params=pltpu.CompilerParams(
            needs_layout_passes=False, use_tc_tiling_on_sc=False),
        scratch_types=[
            pltpu.VMEM((4 * CHUNK,), jnp.float32),
            pltpu.VMEM((4 * CHUNK,), jnp.float32),
            pltpu.VMEM((2 * NSUB, 128), jnp.int32),
            pltpu.VMEM((2 * NSUB, 128), jnp.int32),
            pltpu.VMEM((2 * NSUB, 128), jnp.int32),
            pltpu.VMEM((2 * NSUB, 128), jnp.int32),
            pltpu.VMEM((8, CHUNK), jnp.float32),
            pltpu.VMEM((8, CHUNK), jnp.float32),
            pltpu.VMEM((4 * CHUNK, 8), jnp.float32),
            pltpu.VMEM((4 * CHUNK, 8), jnp.float32),
            pltpu.VMEM((3 * CHUNK,), jnp.float32),
            pltpu.VMEM((3 * CHUNK,), jnp.float32),
            pltpu.VMEM((16,), jnp.float32),
            pltpu.VMEM((BLOCK + 16,), jnp.float32),
            pltpu.VMEM((BLOCK + 16,), jnp.float32),
            pltpu.VMEM((BLOCK + 16,), jnp.float32),
            pltpu.VMEM((BLOCK + 16,), jnp.float32),
            pltpu.VMEM((BLOCK + 16,), jnp.float32),
            pltpu.VMEM((BLOCK + 16,), jnp.float32),
            pltpu.VMEM((BLOCK, 8), jnp.float32),
            pltpu.VMEM((BLOCK, 8), jnp.float32),
            pltpu.SemaphoreType.DMA,
            pltpu.SemaphoreType.DMA,
            pltpu.SemaphoreType.DMA,
            pltpu.SemaphoreType.DMA,
            pltpu.SemaphoreType.DMA,
            pltpu.SemaphoreType.DMA,
            pltpu.SemaphoreType.DMA,
            pltpu.SemaphoreType.DMA,
            pltpu.SemaphoreType.DMA,
            pltpu.SemaphoreType.DMA,
        ],
    )
    return f(im1f, im2f, cfl, m1f, m2f)


def kernel(im1, im2, C, M1, M2):
    out_flat, loss_part, _, _ = _run(
        im1.reshape(N * 3 * R), im2.reshape(N * 3 * R),
        C.reshape(N * 2 * R), M1.reshape(N * R), M2.reshape(N * R))
    out = out_flat.reshape(N, 3, D, D)
    loss = jnp.sum(loss_part) * (0.01 / (N * 2.0 * R * D * D))
    return out, loss
